# baseline (speedup 1.0000x reference)
import jax
import jax.numpy as jnp
from jax.experimental import pallas as pl


def _relu_pallas(x):
    def body(x_ref, o_ref):
        o_ref[...] = jnp.maximum(x_ref[...], 0.0)
    b, v, f = x.shape
    vb = 2048
    vp = ((v + vb - 1) // vb) * vb
    xp = jnp.pad(x, ((0, 0), (0, vp - v), (0, 0)))
    out = pl.pallas_call(
        body,
        grid=(b, vp // vb),
        in_specs=[pl.BlockSpec((1, vb, f), lambda i, j: (i, j, 0))],
        out_specs=pl.BlockSpec((1, vb, f), lambda i, j: (i, j, 0)),
        out_shape=jax.ShapeDtypeStruct((b, vp, f), x.dtype))(xp)
    return out[:, :v, :]


def kernel(x, template, row, col, adj_vals, gc1_W, gc1_b, gc2_W, gc2_b, gc3_W, gc3_b, fc_W, fc_b, mu_W, mu_b, lv_W, lv_b, d1_W, d1_b, d2_W, d2_b, d3_W, d3_b, d4_W, d4_b, eps):
    V = template.shape[0]

    def spmm(h):
        return jax.ops.segment_sum(adj_vals[:, None] * h[col], row, num_segments=V)

    def gconv(h, W, b):
        hh = h @ W + b
        return jax.vmap(spmm)(hh)

    h = _relu_pallas(gconv(x, gc1_W, gc1_b))
    h = _relu_pallas(gconv(h, gc2_W, gc2_b))
    h = _relu_pallas(gconv(h, gc3_W, gc3_b))
    g = h.mean(axis=1)
    g = jax.nn.relu(g @ fc_W + fc_b)
    mu = g @ mu_W + mu_b
    log_var = jnp.clip(g @ lv_W + lv_b, -20.0, 20.0)
    std = jnp.exp(0.5 * log_var)
    z = mu + eps * std
    Bn = z.shape[0]
    z_exp = jnp.broadcast_to(z[:, None, :], (Bn, V, z.shape[-1]))
    t_exp = jnp.broadcast_to(template[None, :, :], (Bn, V, 3))
    hcat = jnp.concatenate([z_exp, t_exp], axis=2)
    d = jax.nn.relu(hcat @ d1_W + d1_b)
    d = jax.nn.relu(d @ d2_W + d2_b)
    d = jax.nn.relu(d @ d3_W + d3_b)
    offsets = d @ d4_W + d4_b
    recon = template[None, :, :] + offsets
    return recon, mu, log_var


# speedup vs baseline: 24.6991x; 24.6991x over previous
import functools
from functools import partial

import jax
import jax.numpy as jnp
from jax import lax
from jax.experimental import pallas as pl
from jax.experimental.pallas import tpu as pltpu
from jax.experimental.pallas import tpu_sc as plsc

VB = 512
SC_W = 8
SC_CH = 1024
SC_TILES = 16
SC_CORES = 2


def _pad_to(n, m):
    return ((n + m - 1) // m) * m


@functools.lru_cache(maxsize=None)
def _make_spmm_sc(v_pad, wtot, nch):
    assert wtot % (2 * SC_W) == 0 and v_pad % SC_TILES == 0
    nblk = wtot // (2 * SC_W)
    rp = v_pad // SC_TILES
    mesh = plsc.VectorSubcoreMesh(core_axis_name="c", subcore_axis_name="s")

    @partial(
        pl.kernel,
        out_type=jax.ShapeDtypeStruct((v_pad, wtot), jnp.float32),
        mesh=mesh,
        scratch_types=[
            pltpu.VMEM((nch, SC_CH), jnp.int32),
            pltpu.VMEM((nch, SC_CH), jnp.int32),
            pltpu.VMEM((SC_CH, SC_W), jnp.float32),
            pltpu.VMEM_SHARED((v_pad, SC_W), jnp.float32),
            pltpu.VMEM_SHARED((v_pad, SC_W), jnp.float32),
        ],
        compiler_params=pltpu.CompilerParams(use_tc_tiling_on_sc=False),
    )
    def spmm(y_hbm, rows_hbm, cols_hbm, zeros_hbm, u_hbm, ridx, cidx, gbuf,
             ysh, osh):
        c = lax.axis_index("c")
        s = lax.axis_index("s")
        r0 = s * rp

        pltpu.sync_copy(rows_hbm.at[s], ridx)
        pltpu.sync_copy(cols_hbm.at[s], cidx)

        def block_body(bi, _):
            c0 = (c * nblk + bi) * SC_W
            pltpu.sync_copy(y_hbm.at[pl.ds(r0, rp), pl.ds(c0, SC_W)],
                            ysh.at[pl.ds(r0, rp)])
            pltpu.sync_copy(zeros_hbm, osh.at[pl.ds(r0, rp)])
            plsc.subcore_barrier()
            for j in range(nch):
                pltpu.sync_copy(ysh.at[cidx.at[j]], gbuf)
                pltpu.sync_copy(gbuf, osh.at[ridx.at[j]], add=True)
            plsc.subcore_barrier()
            pltpu.sync_copy(osh.at[pl.ds(r0, rp)],
                            u_hbm.at[pl.ds(r0, rp), pl.ds(c0, SC_W)])
            return 0
        lax.fori_loop(0, nblk, block_body, 0)

    return spmm


def _spmm_sc(y, rows3d, cols3d, zeros_rp):
    v_pad, wtot = y.shape
    nch = rows3d.shape[1]
    return _make_spmm_sc(v_pad, wtot, nch)(y, rows3d, cols3d, zeros_rp)


def _mm_kernel(x_ref, w_ref, b_ref, deg_ref, o_ref, *, postscale, inrelu):
    x = x_ref[...]
    s = lax.rsqrt(jnp.maximum(deg_ref[...], 1.0))
    if inrelu:
        x = jnp.maximum(x * s, 0.0)
    y = jnp.dot(x, w_ref[...], preferred_element_type=jnp.float32)
    y = y + b_ref[...][None, :]
    if postscale:
        y = y * s
    o_ref[...] = y


def _dense_layer(x, wbig, bbig, deg2, *, inrelu, postscale, interpret=False):
    v_pad, k = x.shape
    n = wbig.shape[1]
    return pl.pallas_call(
        partial(_mm_kernel, postscale=postscale, inrelu=inrelu),
        grid=(v_pad // VB,),
        in_specs=[
            pl.BlockSpec((VB, k), lambda i: (i, 0)),
            pl.BlockSpec((k, n), lambda i: (0, 0)),
            pl.BlockSpec((n,), lambda i: (0,)),
            pl.BlockSpec((VB, 1), lambda i: (i, 0)),
        ],
        out_specs=pl.BlockSpec((VB, n), lambda i: (i, 0)),
        out_shape=jax.ShapeDtypeStruct((v_pad, n), jnp.float32),
        interpret=interpret,
    )(x, wbig, bbig, deg2)


def _head_kernel(u3_ref, deg_ref, fcW_ref, fcb_ref, muW_ref, mub_ref,
                 lvW_ref, lvb_ref, d1Wz_ref, d1b_ref, eps_ref,
                 mu_ref, lv_ref, zd1_ref, acc_ref, *, nb, v, b, hid2):
    i = pl.program_id(0)

    @pl.when(i == 0)
    def _():
        acc_ref[...] = jnp.zeros_like(acc_ref)

    s = lax.rsqrt(jnp.maximum(deg_ref[...], 1.0))
    h = jnp.maximum(u3_ref[...] * s, 0.0)
    rowid = i * VB + lax.broadcasted_iota(jnp.int32, (VB, 1), 0)
    h = jnp.where(rowid < v, h, 0.0)
    acc_ref[...] += jnp.sum(h.reshape(VB, b, hid2), axis=0)

    @pl.when(i == nb - 1)
    def _():
        g = acc_ref[...] / jnp.float32(v)
        g = jnp.maximum(
            jnp.dot(g, fcW_ref[...], preferred_element_type=jnp.float32)
            + fcb_ref[...][None, :], 0.0)
        mu = jnp.dot(g, muW_ref[...], preferred_element_type=jnp.float32) \
            + mub_ref[...][None, :]
        lv = jnp.dot(g, lvW_ref[...], preferred_element_type=jnp.float32) \
            + lvb_ref[...][None, :]
        lv = jnp.clip(lv, -20.0, 20.0)
        z = mu + eps_ref[...] * jnp.exp(0.5 * lv)
        zd1 = jnp.dot(z, d1Wz_ref[...], preferred_element_type=jnp.float32) \
            + d1b_ref[...][None, :]
        mu_ref[...] = jnp.pad(mu, ((0, 8 - b), (0, 128 - mu.shape[1])))
        lv_ref[...] = jnp.pad(lv, ((0, 8 - b), (0, 128 - lv.shape[1])))
        zd1_ref[...] = jnp.pad(zd1, ((0, 8 - b), (0, 0)))


def _head(u3, deg2, fcW, fcb, muW, mub, lvW, lvb, d1Wz, d1b, eps, v, b,
          interpret=False):
    v_pad, w = u3.shape
    hid2 = w // b
    nb = v_pad // VB
    zdim = muW.shape[1]
    full = lambda *shape: pl.BlockSpec(shape, lambda i: (0,) * len(shape))
    mu_p, lv_p, zd1_p = pl.pallas_call(
        partial(_head_kernel, nb=nb, v=v, b=b, hid2=hid2),
        grid=(nb,),
        in_specs=[
            pl.BlockSpec((VB, w), lambda i: (i, 0)),
            pl.BlockSpec((VB, 1), lambda i: (i, 0)),
            full(hid2, 2 * hid2), full(2 * hid2),
            full(2 * hid2, zdim), full(zdim),
            full(2 * hid2, zdim), full(zdim),
            full(zdim, hid2), full(hid2),
            full(b, zdim),
        ],
        out_specs=[full(8, 128), full(8, 128), full(8, hid2)],
        out_shape=[jax.ShapeDtypeStruct((8, 128), jnp.float32),
                   jax.ShapeDtypeStruct((8, 128), jnp.float32),
                   jax.ShapeDtypeStruct((8, hid2), jnp.float32)],
        scratch_shapes=[pltpu.VMEM((b, hid2), jnp.float32)],
        interpret=interpret,
    )(u3, deg2, fcW, fcb, muW, mub, lvW, lvb, d1Wz, d1b, eps)
    return mu_p[:b, :zdim], lv_p[:b, :zdim], zd1_p


def _decoder_kernel(t_ref, zd1_ref, d2W_ref, d2b_ref, d3W_ref, d3b_ref,
                    d4W_ref, d4b_ref, d1Wt_ref, o_ref):
    bi = pl.program_id(0)
    t = t_ref[...]
    d = jnp.dot(t, d1Wt_ref[...], preferred_element_type=jnp.float32)
    zrow = zd1_ref[pl.ds(bi, 1), :]
    d = jnp.maximum(d + zrow, 0.0)
    d = jnp.maximum(
        jnp.dot(d, d2W_ref[...], preferred_element_type=jnp.float32)
        + d2b_ref[...][None, :], 0.0)
    d = jnp.maximum(
        jnp.dot(d, d3W_ref[...], preferred_element_type=jnp.float32)
        + d3b_ref[...][None, :], 0.0)
    off = jnp.dot(d, d4W_ref[...], preferred_element_type=jnp.float32) \
        + d4b_ref[...][None, :]
    o_ref[...] = (off + t)[None]


def _decoder(t8, zd1, d2W, d2b, d3W, d3b, d4W8, d4b8, d1Wt8, b,
             interpret=False):
    v_pad = t8.shape[0]
    nb = v_pad // VB
    hid2 = d2W.shape[0]
    hid = d3W.shape[1]
    full = lambda *shape: pl.BlockSpec(shape, lambda bi, i: (0,) * len(shape))
    return pl.pallas_call(
        _decoder_kernel,
        grid=(b, nb),
        in_specs=[
            pl.BlockSpec((VB, 8), lambda bi, i: (i, 0)),
            pl.BlockSpec((8, hid2), lambda bi, i: (0, 0)),
            full(hid2, hid2), full(hid2),
            full(hid2, hid), full(hid),
            full(hid, 8), full(8),
            full(8, hid2),
        ],
        out_specs=pl.BlockSpec((1, VB, 8), lambda bi, i: (bi, i, 0)),
        out_shape=jax.ShapeDtypeStruct((b, v_pad, 8), jnp.float32),
        interpret=interpret,
    )(t8, zd1, d2W, d2b, d3W, d3b, d4W8, d4b8, d1Wt8)


def kernel(x, template, row, col, adj_vals, gc1_W, gc1_b, gc2_W, gc2_b,
           gc3_W, gc3_b, fc_W, fc_b, mu_W, mu_b, lv_W, lv_b, d1_W, d1_b,
           d2_W, d2_b, d3_W, d3_b, d4_W, d4_b, eps):
    B, V, _ = x.shape
    HID = gc2_W.shape[0]
    E = row.shape[0]
    v_pad = _pad_to(V, VB)
    rp = v_pad // SC_TILES

    et = _pad_to(-(-E // SC_TILES), SC_CH)
    nch = et // SC_CH
    e_pad = et * SC_TILES
    dummy = v_pad - 1
    fill = jnp.full((e_pad - E,), dummy, jnp.int32)
    rows3d = jnp.concatenate([row.astype(jnp.int32), fill])
    cols3d = jnp.concatenate([col.astype(jnp.int32), fill])
    rows3d = rows3d.reshape(SC_TILES, nch, SC_CH)
    cols3d = cols3d.reshape(SC_TILES, nch, SC_CH)
    zeros_rp = jnp.zeros((rp, SC_W), jnp.float32)
    del adj_vals

    ones16 = jnp.ones((v_pad, 2 * SC_W), jnp.float32)
    deg2 = _spmm_sc(ones16, rows3d, cols3d, zeros_rp)[:, :1]

    xt = jnp.pad(x.transpose(1, 0, 2).reshape(V, B * 3),
                 ((0, v_pad - V), (0, 0)))
    eye = jnp.eye(B, dtype=jnp.float32)
    w1big = jnp.einsum("ab,ch->acbh", eye, gc1_W).reshape(B * 3, B * HID)
    w2big = jnp.einsum("ab,ch->acbh", eye, gc2_W).reshape(B * HID, B * HID)
    w3big = jnp.einsum("ab,ch->acbh", eye, gc3_W).reshape(B * HID,
                                                          B * 2 * HID)
    b1big = jnp.tile(gc1_b, B)
    b2big = jnp.tile(gc2_b, B)
    b3big = jnp.tile(gc3_b, B)

    y1 = _dense_layer(xt, w1big, b1big, deg2, inrelu=False, postscale=True)
    u1 = _spmm_sc(y1, rows3d, cols3d, zeros_rp)
    y2 = _dense_layer(u1, w2big, b2big, deg2, inrelu=True, postscale=True)
    u2 = _spmm_sc(y2, rows3d, cols3d, zeros_rp)
    y3 = _dense_layer(u2, w3big, b3big, deg2, inrelu=True, postscale=True)
    u3 = _spmm_sc(y3, rows3d, cols3d, zeros_rp)

    mu, log_var, zd1 = _head(u3, deg2, fc_W, fc_b, mu_W, mu_b, lv_W, lv_b,
                             d1_W[:mu_W.shape[1]], d1_b, eps, V, B)

    t8 = jnp.pad(template, ((0, v_pad - V), (0, 8 - 3)))
    d1Wt8 = jnp.pad(d1_W[mu_W.shape[1]:], ((0, 8 - 3), (0, 0)))
    d4W8 = jnp.pad(d4_W, ((0, 0), (0, 8 - 3)))
    d4b8 = jnp.pad(d4_b, ((0, 8 - 3),))
    recon8 = _decoder(t8, zd1, d2_W, d2_b, d3_W, d3_b, d4W8, d4b8, d1Wt8, B)
    recon = recon8[:, :V, :3]
    return recon, mu, log_var


# speedup vs baseline: 26.5341x; 1.0743x over previous
import functools
from functools import partial

import jax
import jax.numpy as jnp
from jax import lax
from jax.experimental import pallas as pl
from jax.experimental.pallas import tpu as pltpu
from jax.experimental.pallas import tpu_sc as plsc

VB = 512
SC_W = 8
SC_CH = 1024
SC_TILES = 16
SC_CORES = 2


def _pad_to(n, m):
    return ((n + m - 1) // m) * m


@functools.lru_cache(maxsize=None)
def _make_spmm_sc(v_pad, wtot, nch):
    assert wtot % (2 * SC_W) == 0 and v_pad % SC_TILES == 0
    nblk = wtot // (2 * SC_W)
    rp = v_pad // SC_TILES
    mesh = plsc.VectorSubcoreMesh(core_axis_name="c", subcore_axis_name="s")

    @partial(
        pl.kernel,
        out_type=jax.ShapeDtypeStruct((v_pad, wtot), jnp.float32),
        mesh=mesh,
        scratch_types=[
            pltpu.VMEM((nch, SC_CH), jnp.int32),
            pltpu.VMEM((nch, SC_CH), jnp.int32),
            pltpu.VMEM((SC_CH, SC_W), jnp.float32),
            pltpu.VMEM((SC_CH, SC_W), jnp.float32),
            pltpu.VMEM_SHARED((v_pad, SC_W), jnp.float32),
            pltpu.VMEM_SHARED((v_pad, SC_W), jnp.float32),
            pltpu.SemaphoreType.DMA,
            pltpu.SemaphoreType.DMA,
        ],
        compiler_params=pltpu.CompilerParams(use_tc_tiling_on_sc=False),
    )
    def spmm(y_hbm, rows_hbm, cols_hbm, zeros_hbm, u_hbm, ridx, cidx, gb0,
             gb1, ysh, osh, sem0, sem1):
        c = lax.axis_index("c")
        s = lax.axis_index("s")
        r0 = s * rp
        gbufs = (gb0, gb1)
        sems = (sem0, sem1)

        pltpu.sync_copy(rows_hbm.at[s], ridx)
        pltpu.sync_copy(cols_hbm.at[s], cidx)

        def block_body(bi, _):
            c0 = (c * nblk + bi) * SC_W
            pltpu.sync_copy(y_hbm.at[pl.ds(r0, rp), pl.ds(c0, SC_W)],
                            ysh.at[pl.ds(r0, rp)])
            pltpu.sync_copy(zeros_hbm, osh.at[pl.ds(r0, rp)])
            plsc.subcore_barrier()
            descs = [None, None]
            descs[0] = pltpu.async_copy(ysh.at[cidx.at[0]], gb0, sem0)
            for j in range(nch):
                if j + 1 < nch:
                    k = (j + 1) % 2
                    descs[k] = pltpu.async_copy(ysh.at[cidx.at[j + 1]],
                                                gbufs[k], sems[k])
                descs[j % 2].wait()
                pltpu.sync_copy(gbufs[j % 2], osh.at[ridx.at[j]], add=True)
            plsc.subcore_barrier()
            pltpu.sync_copy(osh.at[pl.ds(r0, rp)],
                            u_hbm.at[pl.ds(r0, rp), pl.ds(c0, SC_W)])
            return 0
        lax.fori_loop(0, nblk, block_body, 0)

    return spmm


def _spmm_sc(y, rows3d, cols3d, zeros_rp):
    v_pad, wtot = y.shape
    nch = rows3d.shape[1]
    return _make_spmm_sc(v_pad, wtot, nch)(y, rows3d, cols3d, zeros_rp)


def _mm_kernel(x_ref, w_ref, b_ref, deg_ref, o_ref, *, postscale, inrelu):
    x = x_ref[...]
    s = lax.rsqrt(jnp.maximum(deg_ref[...], 1.0))
    if inrelu:
        x = jnp.maximum(x * s, 0.0)
    y = jnp.dot(x, w_ref[...], preferred_element_type=jnp.float32)
    y = y + b_ref[...][None, :]
    if postscale:
        y = y * s
    o_ref[...] = y


def _dense_layer(x, wbig, bbig, deg2, *, inrelu, postscale, interpret=False):
    v_pad, k = x.shape
    n = wbig.shape[1]
    return pl.pallas_call(
        partial(_mm_kernel, postscale=postscale, inrelu=inrelu),
        grid=(v_pad // VB,),
        in_specs=[
            pl.BlockSpec((VB, k), lambda i: (i, 0)),
            pl.BlockSpec((k, n), lambda i: (0, 0)),
            pl.BlockSpec((n,), lambda i: (0,)),
            pl.BlockSpec((VB, 1), lambda i: (i, 0)),
        ],
        out_specs=pl.BlockSpec((VB, n), lambda i: (i, 0)),
        out_shape=jax.ShapeDtypeStruct((v_pad, n), jnp.float32),
        interpret=interpret,
    )(x, wbig, bbig, deg2)


def _head_kernel(u3_ref, deg_ref, fcW_ref, fcb_ref, muW_ref, mub_ref,
                 lvW_ref, lvb_ref, d1Wz_ref, d1b_ref, eps_ref,
                 mu_ref, lv_ref, zd1_ref, acc_ref, *, nb, v, b, hid2):
    i = pl.program_id(0)

    @pl.when(i == 0)
    def _():
        acc_ref[...] = jnp.zeros_like(acc_ref)

    s = lax.rsqrt(jnp.maximum(deg_ref[...], 1.0))
    h = jnp.maximum(u3_ref[...] * s, 0.0)
    rowid = i * VB + lax.broadcasted_iota(jnp.int32, (VB, 1), 0)
    h = jnp.where(rowid < v, h, 0.0)
    acc_ref[...] += jnp.sum(h.reshape(VB, b, hid2), axis=0)

    @pl.when(i == nb - 1)
    def _():
        g = acc_ref[...] / jnp.float32(v)
        g = jnp.maximum(
            jnp.dot(g, fcW_ref[...], preferred_element_type=jnp.float32)
            + fcb_ref[...][None, :], 0.0)
        mu = jnp.dot(g, muW_ref[...], preferred_element_type=jnp.float32) \
            + mub_ref[...][None, :]
        lv = jnp.dot(g, lvW_ref[...], preferred_element_type=jnp.float32) \
            + lvb_ref[...][None, :]
        lv = jnp.clip(lv, -20.0, 20.0)
        z = mu + eps_ref[...] * jnp.exp(0.5 * lv)
        zd1 = jnp.dot(z, d1Wz_ref[...], preferred_element_type=jnp.float32) \
            + d1b_ref[...][None, :]
        mu_ref[...] = jnp.pad(mu, ((0, 8 - b), (0, 128 - mu.shape[1])))
        lv_ref[...] = jnp.pad(lv, ((0, 8 - b), (0, 128 - lv.shape[1])))
        zd1_ref[...] = jnp.pad(zd1, ((0, 8 - b), (0, 0)))


def _head(u3, deg2, fcW, fcb, muW, mub, lvW, lvb, d1Wz, d1b, eps, v, b,
          interpret=False):
    v_pad, w = u3.shape
    hid2 = w // b
    nb = v_pad // VB
    zdim = muW.shape[1]
    full = lambda *shape: pl.BlockSpec(shape, lambda i: (0,) * len(shape))
    mu_p, lv_p, zd1_p = pl.pallas_call(
        partial(_head_kernel, nb=nb, v=v, b=b, hid2=hid2),
        grid=(nb,),
        in_specs=[
            pl.BlockSpec((VB, w), lambda i: (i, 0)),
            pl.BlockSpec((VB, 1), lambda i: (i, 0)),
            full(hid2, 2 * hid2), full(2 * hid2),
            full(2 * hid2, zdim), full(zdim),
            full(2 * hid2, zdim), full(zdim),
            full(zdim, hid2), full(hid2),
            full(b, zdim),
        ],
        out_specs=[full(8, 128), full(8, 128), full(8, hid2)],
        out_shape=[jax.ShapeDtypeStruct((8, 128), jnp.float32),
                   jax.ShapeDtypeStruct((8, 128), jnp.float32),
                   jax.ShapeDtypeStruct((8, hid2), jnp.float32)],
        scratch_shapes=[pltpu.VMEM((b, hid2), jnp.float32)],
        interpret=interpret,
    )(u3, deg2, fcW, fcb, muW, mub, lvW, lvb, d1Wz, d1b, eps)
    return mu_p[:b, :zdim], lv_p[:b, :zdim], zd1_p


def _decoder_kernel(t_ref, zd1_ref, d2W_ref, d2b_ref, d3W_ref, d3b_ref,
                    d4W_ref, d4b_ref, d1Wt_ref, o_ref):
    bi = pl.program_id(0)
    t = t_ref[...]
    d = jnp.dot(t, d1Wt_ref[...], preferred_element_type=jnp.float32)
    zrow = zd1_ref[pl.ds(bi, 1), :]
    d = jnp.maximum(d + zrow, 0.0)
    d = jnp.maximum(
        jnp.dot(d, d2W_ref[...], preferred_element_type=jnp.float32)
        + d2b_ref[...][None, :], 0.0)
    d = jnp.maximum(
        jnp.dot(d, d3W_ref[...], preferred_element_type=jnp.float32)
        + d3b_ref[...][None, :], 0.0)
    off = jnp.dot(d, d4W_ref[...], preferred_element_type=jnp.float32) \
        + d4b_ref[...][None, :]
    o_ref[...] = (off + t)[None]


def _decoder(t8, zd1, d2W, d2b, d3W, d3b, d4W8, d4b8, d1Wt8, b,
             interpret=False):
    v_pad = t8.shape[0]
    nb = v_pad // VB
    hid2 = d2W.shape[0]
    hid = d3W.shape[1]
    full = lambda *shape: pl.BlockSpec(shape, lambda bi, i: (0,) * len(shape))
    return pl.pallas_call(
        _decoder_kernel,
        grid=(b, nb),
        in_specs=[
            pl.BlockSpec((VB, 8), lambda bi, i: (i, 0)),
            pl.BlockSpec((8, hid2), lambda bi, i: (0, 0)),
            full(hid2, hid2), full(hid2),
            full(hid2, hid), full(hid),
            full(hid, 8), full(8),
            full(8, hid2),
        ],
        out_specs=pl.BlockSpec((1, VB, 8), lambda bi, i: (bi, i, 0)),
        out_shape=jax.ShapeDtypeStruct((b, v_pad, 8), jnp.float32),
        interpret=interpret,
    )(t8, zd1, d2W, d2b, d3W, d3b, d4W8, d4b8, d1Wt8)


def kernel(x, template, row, col, adj_vals, gc1_W, gc1_b, gc2_W, gc2_b,
           gc3_W, gc3_b, fc_W, fc_b, mu_W, mu_b, lv_W, lv_b, d1_W, d1_b,
           d2_W, d2_b, d3_W, d3_b, d4_W, d4_b, eps):
    B, V, _ = x.shape
    HID = gc2_W.shape[0]
    E = row.shape[0]
    v_pad = _pad_to(V, VB)
    rp = v_pad // SC_TILES

    et = _pad_to(-(-E // SC_TILES), SC_CH)
    nch = et // SC_CH
    e_pad = et * SC_TILES
    dummy = v_pad - 1
    fill = jnp.full((e_pad - E,), dummy, jnp.int32)
    rows3d = jnp.concatenate([row.astype(jnp.int32), fill])
    cols3d = jnp.concatenate([col.astype(jnp.int32), fill])
    rows3d = rows3d.reshape(SC_TILES, nch, SC_CH)
    cols3d = cols3d.reshape(SC_TILES, nch, SC_CH)
    zeros_rp = jnp.zeros((rp, SC_W), jnp.float32)
    del adj_vals

    ones16 = jnp.ones((v_pad, 2 * SC_W), jnp.float32)
    deg2 = _spmm_sc(ones16, rows3d, cols3d, zeros_rp)[:, :1]

    xt = jnp.pad(x.transpose(1, 0, 2).reshape(V, B * 3),
                 ((0, v_pad - V), (0, 0)))
    eye = jnp.eye(B, dtype=jnp.float32)
    w1big = jnp.einsum("ab,ch->acbh", eye, gc1_W).reshape(B * 3, B * HID)
    w2big = jnp.einsum("ab,ch->acbh", eye, gc2_W).reshape(B * HID, B * HID)
    w3big = jnp.einsum("ab,ch->acbh", eye, gc3_W).reshape(B * HID,
                                                          B * 2 * HID)
    b1big = jnp.tile(gc1_b, B)
    b2big = jnp.tile(gc2_b, B)
    b3big = jnp.tile(gc3_b, B)

    y1 = _dense_layer(xt, w1big, b1big, deg2, inrelu=False, postscale=True)
    u1 = _spmm_sc(y1, rows3d, cols3d, zeros_rp)
    y2 = _dense_layer(u1, w2big, b2big, deg2, inrelu=True, postscale=True)
    u2 = _spmm_sc(y2, rows3d, cols3d, zeros_rp)
    y3 = _dense_layer(u2, w3big, b3big, deg2, inrelu=True, postscale=True)
    u3 = _spmm_sc(y3, rows3d, cols3d, zeros_rp)

    mu, log_var, zd1 = _head(u3, deg2, fc_W, fc_b, mu_W, mu_b, lv_W, lv_b,
                             d1_W[:mu_W.shape[1]], d1_b, eps, V, B)

    t8 = jnp.pad(template, ((0, v_pad - V), (0, 8 - 3)))
    d1Wt8 = jnp.pad(d1_W[mu_W.shape[1]:], ((0, 8 - 3), (0, 0)))
    d4W8 = jnp.pad(d4_W, ((0, 0), (0, 8 - 3)))
    d4b8 = jnp.pad(d4_b, ((0, 8 - 3),))
    recon8 = _decoder(t8, zd1, d2_W, d2_b, d3_W, d3_b, d4W8, d4b8, d1Wt8, B)
    recon = recon8[:, :V, :3]
    return recon, mu, log_var
